# baseline (speedup 1.0000x reference)
import jax
import jax.numpy as jnp
from jax.experimental import pallas as pl
from jax.experimental.pallas import tpu as pltpu

E = 8
TOPK = 2
D = 1024
F = 768
FS = 1536
T = 2048

TM = 256
CH = 128
NP = T * TOPK


def _shared_router_kernel(x_ref, wr_ref, wsgu_ref, wsd_ref, wsg_ref,
                          sh_ref, aux_ref, cnt_ref):
    i = pl.program_id(0)

    @pl.when(i == 0)
    def _init():
        cnt_ref[...] = jnp.zeros_like(cnt_ref)

    x32 = x_ref[...]
    xb = x32.astype(jnp.bfloat16)
    gu = jnp.dot(xb, wsgu_ref[...], preferred_element_type=jnp.float32)
    g = gu[:, :FS]
    u = gu[:, FS:]
    h = (u * g * jax.nn.sigmoid(g)).astype(jnp.bfloat16)
    sh = jnp.dot(h, wsd_ref[...], preferred_element_type=jnp.float32)
    sg_logit = jnp.sum(x32 * wsg_ref[...].reshape(1, D), axis=1, keepdims=True)
    sh_ref[...] = sh * jax.nn.sigmoid(sg_logit)

    logits = jnp.dot(xb, wr_ref[...],
                     preferred_element_type=jnp.float32)
    cols = jax.lax.broadcasted_iota(jnp.int32, (TM, E), 1)
    m1 = jnp.max(logits, axis=1, keepdims=True)
    i1 = jnp.min(jnp.where(logits == m1, cols, E), axis=1, keepdims=True)
    masked = jnp.where(cols == i1, -jnp.inf, logits)
    m2 = jnp.max(masked, axis=1, keepdims=True)
    i2 = jnp.min(jnp.where(masked == m2, cols, E), axis=1, keepdims=True)
    w1 = jax.nn.sigmoid(m1 - m2)

    o1 = (cols == i1).astype(jnp.float32)
    o2 = (cols == i2).astype(jnp.float32)
    s = o1 + o2
    tri = (jax.lax.broadcasted_iota(jnp.int32, (TM, TM), 0)
           > jax.lax.broadcasted_iota(jnp.int32, (TM, TM), 1)
           ).astype(jnp.float32)
    excl = jnp.dot(tri, s, preferred_element_type=jnp.float32)
    cnt = cnt_ref[...]
    base = excl + cnt
    grank1 = jnp.sum(base * o1, axis=1, keepdims=True)
    grank2 = jnp.sum((base + o1) * o2, axis=1, keepdims=True)
    cnt_new = cnt + jnp.sum(s, axis=0, keepdims=True)
    cnt_ref[...] = cnt_new

    lane = jax.lax.broadcasted_iota(jnp.int32, (TM, 128), 1)
    lane_r = jax.lax.broadcasted_iota(jnp.int32, (1, 128), 1)
    cnt128 = jnp.zeros((1, 128), jnp.float32)
    for ei in range(E):
        cnt128 = jnp.where(lane_r == 8 + ei, cnt_new[0, ei], cnt128)
    aux = jnp.where(lane == 0, i1.astype(jnp.float32),
          jnp.where(lane == 1, i2.astype(jnp.float32),
          jnp.where(lane == 2, w1,
          jnp.where(lane == 3, grank1,
          jnp.where(lane == 4, grank2, cnt128)))))
    aux_ref[...] = aux


def _grouped_mlp_kernel(offs, x_ref, w_ref, wgu_ref, wd_ref, out_ref):
    e = pl.program_id(0)
    start = offs[e]
    end = offs[e + 1]
    c0 = start // CH
    c1 = pl.cdiv(end, CH)
    wgu = wgu_ref[0]
    wd = wd_ref[0]

    def body(c, _):
        base = c * CH
        xb = x_ref[pl.ds(base, CH), :]
        gu = jnp.dot(xb, wgu, preferred_element_type=jnp.float32)
        g = gu[:, :F]
        u = gu[:, F:]
        h = (u * g * jax.nn.sigmoid(g)).astype(jnp.bfloat16)
        y = jnp.dot(h, wd, preferred_element_type=jnp.float32)
        y = y * w_ref[pl.ds(base, CH), :]
        rows = base + jax.lax.broadcasted_iota(jnp.int32, (CH, 1), 0)
        mask = (rows >= start) & (rows < end)
        cur = out_ref[pl.ds(base, CH), :]
        out_ref[pl.ds(base, CH), :] = jnp.where(
            mask, y.astype(jnp.bfloat16), cur)
        return 0

    jax.lax.fori_loop(c0, c1, body, 0)


def kernel(hidden_states, W_router, W_gate_up, W_down, Ws_gate_up, Ws_down,
           w_shared_gate):
    x = hidden_states
    xb16 = x.astype(jnp.bfloat16)
    wgu16 = W_gate_up.astype(jnp.bfloat16)
    wd16 = W_down.astype(jnp.bfloat16)
    wsgu16 = Ws_gate_up.astype(jnp.bfloat16)
    wsd16 = Ws_down.astype(jnp.bfloat16)

    shared, aux = pl.pallas_call(
        _shared_router_kernel,
        grid=(T // TM,),
        in_specs=[
            pl.BlockSpec((TM, D), lambda i: (i, 0)),
            pl.BlockSpec((D, E), lambda i: (0, 0)),
            pl.BlockSpec((D, 2 * FS), lambda i: (0, 0)),
            pl.BlockSpec((FS, D), lambda i: (0, 0)),
            pl.BlockSpec((D, 1), lambda i: (0, 0)),
        ],
        out_specs=[
            pl.BlockSpec((TM, D), lambda i: (i, 0)),
            pl.BlockSpec((TM, 128), lambda i: (i, 0)),
        ],
        out_shape=[
            jax.ShapeDtypeStruct((T, D), jnp.float32),
            jax.ShapeDtypeStruct((T, 128), jnp.float32),
        ],
        scratch_shapes=[pltpu.VMEM((1, E), jnp.float32)],
        compiler_params=pltpu.CompilerParams(
            dimension_semantics=("arbitrary",)),
    )(x, W_router.astype(jnp.bfloat16), wsgu16, wsd16, w_shared_gate)

    i1 = aux[:, 0].astype(jnp.int32)
    i2 = aux[:, 1].astype(jnp.int32)
    w1 = aux[:, 2]
    flat_e = jnp.stack([i1, i2], axis=1).reshape(-1)
    flat_w = jnp.stack([w1, 1.0 - w1], axis=1).reshape(-1)
    grank = jnp.stack([aux[:, 3], aux[:, 4]], axis=1).reshape(-1)
    grank = grank.astype(jnp.int32)
    counts = aux[T - 1, 8:8 + E].astype(jnp.int32)
    offs = jnp.concatenate(
        [jnp.zeros(1, jnp.int32), jnp.cumsum(counts).astype(jnp.int32)])

    dest = offs[flat_e] + grank
    sorted_pair = jnp.zeros((NP,), jnp.int32).at[dest].set(
        jnp.arange(NP, dtype=jnp.int32), unique_indices=True)
    sorted_tok = sorted_pair // TOPK
    sorted_w = flat_w[sorted_pair].reshape(-1, 1)
    x_sorted = xb16[sorted_tok]

    out_sorted = pl.pallas_call(
        _grouped_mlp_kernel,
        grid_spec=pltpu.PrefetchScalarGridSpec(
            num_scalar_prefetch=1,
            grid=(E,),
            in_specs=[
                pl.BlockSpec((NP, D), lambda e, offs: (0, 0)),
                pl.BlockSpec((NP, 1), lambda e, offs: (0, 0)),
                pl.BlockSpec((1, D, 2 * F), lambda e, offs: (e, 0, 0)),
                pl.BlockSpec((1, F, D), lambda e, offs: (e, 0, 0)),
            ],
            out_specs=pl.BlockSpec((NP, D), lambda e, offs: (0, 0)),
        ),
        out_shape=jax.ShapeDtypeStruct((NP, D), jnp.bfloat16),
        compiler_params=pltpu.CompilerParams(
            dimension_semantics=("arbitrary",)),
    )(offs, x_sorted, sorted_w, wgu16, wd16)

    routed = out_sorted[dest].astype(jnp.float32).reshape(T, TOPK, D)
    return shared + routed.sum(axis=1)


# speedup vs baseline: 1.4399x; 1.4399x over previous
import jax
import jax.numpy as jnp
from jax.experimental import pallas as pl
from jax.experimental.pallas import tpu as pltpu

E = 8
TOPK = 2
D = 1024
F = 768
FS = 1536
T = 2048

TM = 256
CH = 128
NP = T * TOPK


def _moe_kernel(x_ref, wr_ref, wsgu_ref, wsd_ref, wsg_ref, wgu_hbm, wd_hbm,
                out_ref, os_ref, wgu_buf, wd_buf, sem_gu, sem_d):

    def start_copy(e, slot):
        pltpu.make_async_copy(wgu_hbm.at[e], wgu_buf.at[slot],
                              sem_gu.at[slot]).start()
        pltpu.make_async_copy(wd_hbm.at[e], wd_buf.at[slot],
                              sem_d.at[slot]).start()

    def wait_copy(e, slot):
        pltpu.make_async_copy(wgu_hbm.at[e], wgu_buf.at[slot],
                              sem_gu.at[slot]).wait()
        pltpu.make_async_copy(wd_hbm.at[e], wd_buf.at[slot],
                              sem_d.at[slot]).wait()

    start_copy(0, 0)

    tri = (jax.lax.broadcasted_iota(jnp.int32, (TM, TM), 0)
           > jax.lax.broadcasted_iota(jnp.int32, (TM, TM), 1)
           ).astype(jnp.float32)
    cols = jax.lax.broadcasted_iota(jnp.int32, (TM, E), 1)
    wsg_row = wsg_ref[...].reshape(1, D)
    cnt = jnp.zeros((1, E), jnp.float32)
    i1s, i2s, w1s, g1s, g2s = [], [], [], [], []
    for b in range(T // TM):
        xb = x_ref[pl.ds(b * TM, TM), :]
        gu = jnp.dot(xb, wsgu_ref[...], preferred_element_type=jnp.float32)
        g = gu[:, :FS]
        u = gu[:, FS:]
        h = (u * g * jax.nn.sigmoid(g)).astype(jnp.bfloat16)
        sh = jnp.dot(h, wsd_ref[...], preferred_element_type=jnp.float32)
        sg_logit = jnp.sum(xb.astype(jnp.float32) * wsg_row, axis=1,
                           keepdims=True)
        out_ref[pl.ds(b * TM, TM), :] = sh * jax.nn.sigmoid(sg_logit)

        logits = jnp.dot(xb, wr_ref[...],
                         preferred_element_type=jnp.float32)
        m1 = jnp.max(logits, axis=1, keepdims=True)
        i1 = jnp.min(jnp.where(logits == m1, cols, E), axis=1, keepdims=True)
        masked = jnp.where(cols == i1, -jnp.inf, logits)
        m2 = jnp.max(masked, axis=1, keepdims=True)
        i2 = jnp.min(jnp.where(masked == m2, cols, E), axis=1, keepdims=True)
        w1 = jax.nn.sigmoid(m1 - m2)

        o1 = (cols == i1).astype(jnp.float32)
        o2 = (cols == i2).astype(jnp.float32)
        s = o1 + o2
        excl = jnp.dot(tri, s, preferred_element_type=jnp.float32)
        base = excl + cnt
        g1 = jnp.sum(base * o1, axis=1, keepdims=True)
        g2 = jnp.sum((base + o1) * o2, axis=1, keepdims=True)
        cnt = cnt + jnp.sum(s, axis=0, keepdims=True)
        i1s.append(i1.astype(jnp.float32))
        i2s.append(i2.astype(jnp.float32))
        w1s.append(w1)
        g1s.append(g1)
        g2s.append(g2)

    i1c = jnp.concatenate(i1s, axis=0)
    i2c = jnp.concatenate(i2s, axis=0)
    w1c = jnp.concatenate(w1s, axis=0)
    g1c = jnp.concatenate(g1s, axis=0)
    g2c = jnp.concatenate(g2s, axis=0)

    tri8 = (jax.lax.broadcasted_iota(jnp.int32, (E, E), 0)
            < jax.lax.broadcasted_iota(jnp.int32, (E, E), 1)
            ).astype(jnp.float32)
    offs = jnp.dot(cnt, tri8, preferred_element_type=jnp.float32,
                   precision=jax.lax.Precision.HIGHEST)
    off1 = jnp.zeros_like(g1c)
    off2 = jnp.zeros_like(g2c)
    for e in range(E):
        off1 = jnp.where(i1c == e, offs[0, e], off1)
        off2 = jnp.where(i2c == e, offs[0, e], off2)
    d1 = off1 + g1c
    d2 = off2 + g2c

    offs_i = offs.astype(jnp.int32)
    cnt_i = cnt.astype(jnp.int32)

    for e in range(E):
        slot = e % 2
        if e + 1 < E:
            start_copy(e + 1, 1 - slot)
        wait_copy(e, slot)
        wgu = wgu_buf[slot]
        wd = wd_buf[slot]
        start = offs_i[0, e]
        end = start + cnt_i[0, e]

        def chunk(c, _, wgu=wgu, wd=wd, start=start, end=end):
            sbase = c * CH
            sf = (sbase + jax.lax.broadcasted_iota(jnp.int32, (1, CH), 1)
                  ).astype(jnp.float32)
            sel = ((d1 == sf) | (d2 == sf)).astype(jnp.bfloat16)
            xg = jax.lax.dot_general(
                sel, x_ref[...], (((0,), (0,)), ((), ())),
                preferred_element_type=jnp.float32).astype(jnp.bfloat16)
            gu = jnp.dot(xg, wgu, preferred_element_type=jnp.float32)
            g = gu[:, :F]
            u = gu[:, F:]
            h = (u * g * jax.nn.sigmoid(g)).astype(jnp.bfloat16)
            y = jnp.dot(h, wd, preferred_element_type=jnp.float32)
            rows = sbase + jax.lax.broadcasted_iota(jnp.int32, (CH, 1), 0)
            mask = (rows >= start) & (rows < end)
            cur = os_ref[pl.ds(sbase, CH), :]
            os_ref[pl.ds(sbase, CH), :] = jnp.where(
                mask, y.astype(jnp.bfloat16), cur)
            return 0

        jax.lax.fori_loop(start // CH, pl.cdiv(end, CH), chunk, 0)

    s4 = jax.lax.broadcasted_iota(jnp.int32, (1, NP), 1).astype(jnp.float32)
    for tb in range(T // CH):
        d1c = d1[tb * CH:(tb + 1) * CH, :]
        d2c = d2[tb * CH:(tb + 1) * CH, :]
        wc = w1c[tb * CH:(tb + 1) * CH, :]
        un = (jnp.where(d1c == s4, wc, 0.0)
              + jnp.where(d2c == s4, 1.0 - wc, 0.0)).astype(jnp.bfloat16)
        routed = jnp.dot(un, os_ref[...], preferred_element_type=jnp.float32)
        out_ref[pl.ds(tb * CH, CH), :] += routed


def kernel(hidden_states, W_router, W_gate_up, W_down, Ws_gate_up, Ws_down,
           w_shared_gate):
    xb16 = hidden_states.astype(jnp.bfloat16)
    out = pl.pallas_call(
        _moe_kernel,
        grid=(1,),
        in_specs=[
            pl.BlockSpec((T, D), lambda i: (0, 0)),
            pl.BlockSpec((D, E), lambda i: (0, 0)),
            pl.BlockSpec((D, 2 * FS), lambda i: (0, 0)),
            pl.BlockSpec((FS, D), lambda i: (0, 0)),
            pl.BlockSpec((D, 1), lambda i: (0, 0)),
            pl.BlockSpec(memory_space=pltpu.MemorySpace.HBM),
            pl.BlockSpec(memory_space=pltpu.MemorySpace.HBM),
        ],
        out_specs=pl.BlockSpec((T, D), lambda i: (0, 0)),
        out_shape=jax.ShapeDtypeStruct((T, D), jnp.float32),
        scratch_shapes=[
            pltpu.VMEM((NP, D), jnp.bfloat16),
            pltpu.VMEM((2, D, 2 * F), jnp.bfloat16),
            pltpu.VMEM((2, F, D), jnp.bfloat16),
            pltpu.SemaphoreType.DMA((2,)),
            pltpu.SemaphoreType.DMA((2,)),
        ],
        compiler_params=pltpu.CompilerParams(
            dimension_semantics=("arbitrary",)),
    )(xb16, W_router.astype(jnp.bfloat16), Ws_gate_up.astype(jnp.bfloat16),
      Ws_down.astype(jnp.bfloat16), w_shared_gate,
      W_gate_up.astype(jnp.bfloat16), W_down.astype(jnp.bfloat16))
    return out
